# baseline (speedup 1.0000x reference)
import functools

import jax
import jax.numpy as jnp
from jax import lax
from jax.experimental import pallas as pl
from jax.experimental.pallas import tpu as pltpu
from jax.experimental.pallas import tpu_sc as plsc

BATCH = 4
SEQ = 4096
VOCAB = 100000
K = 2
TOKENS = BATCH * SEQ
NUM_WORKERS = 32
TPW = TOKENS // NUM_WORKERS

_mesh = plsc.VectorSubcoreMesh(core_axis_name="c", subcore_axis_name="s")


@functools.partial(
    pl.kernel,
    mesh=_mesh,
    compiler_params=pltpu.CompilerParams(
        use_tc_tiling_on_sc=False, needs_layout_passes=False
    ),
    out_type=(
        jax.ShapeDtypeStruct((TOKENS,), jnp.uint32),
        jax.ShapeDtypeStruct((TOKENS,), jnp.uint32),
    ),
    scratch_types=[
        pltpu.VMEM((TPW,), jnp.int32),
        pltpu.VMEM((TPW,), jnp.uint32),
        pltpu.VMEM((TPW,), jnp.uint32),
        pltpu.SemaphoreType.DMA,
        pltpu.SemaphoreType.DMA,
        pltpu.SemaphoreType.DMA,
    ],
)
def _hash_gather(
    idx_hbm, tab0_hbm, tab1_hbm, out0_hbm, out1_hbm,
    idx_v, r0_v, r1_v, sem_i, sem0, sem1,
):
    wid = lax.axis_index("s") * 2 + lax.axis_index("c")
    base = wid * TPW
    idx_cp = pltpu.make_async_copy(idx_hbm.at[pl.ds(base, TPW)], idx_v, sem_i)
    idx_cp.start()
    g0 = pltpu.make_async_copy(tab0_hbm.at[idx_v], r0_v, sem0)
    g1 = pltpu.make_async_copy(tab1_hbm.at[idx_v], r1_v, sem1)
    idx_cp.wait()
    g0.start()
    g1.start()
    g0.wait()
    o0 = pltpu.make_async_copy(r0_v, out0_hbm.at[pl.ds(base, TPW)], sem0)
    o0.start()
    g1.wait()
    o1 = pltpu.make_async_copy(r1_v, out1_hbm.at[pl.ds(base, TPW)], sem1)
    o1.start()
    o0.wait()
    o1.wait()


def kernel(input, hash):
    idx = input.astype(jnp.int32).reshape(TOKENS)
    tab0 = hash[:, 0].astype(jnp.uint32)
    tab1 = hash[:, 1].astype(jnp.uint32)
    r0, r1 = _hash_gather(idx, tab0, tab1)
    h0 = r0.astype(hash.dtype).reshape(BATCH, SEQ)
    h1 = r1.astype(hash.dtype).reshape(BATCH, SEQ)
    return jnp.stack([h0, h1], axis=-1)
